# baseline (speedup 1.0000x reference)
import functools
import math

import jax
import jax.numpy as jnp
from jax import lax
from jax.experimental import pallas as pl
from jax.experimental.pallas import tpu as pltpu
from jax.experimental.pallas import tpu_sc as plsc

D_MODEL = 64
SCALE = math.sqrt(D_MODEL)
LANES = 16
NC, NS = 2, 16
NW = NC * NS
B_TOTAL = 4096 * 200
BW = B_TOTAL // NW
IDXC = 128
CHUNK = 512
NG = CHUNK // IDXC
NCH = BW // CHUNK
IDX_ROWS = BW // IDXC

@functools.cache
def _build():
    mesh = plsc.VectorSubcoreMesh(
        core_axis_name="c", subcore_axis_name="s", num_cores=NC, num_subcores=NS
    )
    return functools.partial(
        pl.kernel,
        out_type=jax.ShapeDtypeStruct((B_TOTAL, D_MODEL), jnp.float32),
        mesh=mesh,
        scratch_types=[
            pltpu.VMEM((IDX_ROWS, IDXC), jnp.int32),
            pltpu.VMEM((CHUNK, D_MODEL), jnp.float32),
            pltpu.VMEM((CHUNK, D_MODEL), jnp.float32),
            pltpu.SemaphoreType.DMA,
            pltpu.SemaphoreType.DMA,
            pltpu.SemaphoreType.DMA,
            pltpu.SemaphoreType.DMA,
        ],
        compiler_params=pltpu.CompilerParams(use_tc_tiling_on_sc=False),
    )(_emb_body)


def _emb_body(x_hbm, table_hbm, out_hbm, idx_v, buf0, buf1, g0, g1, s0, s1):
    wid = lax.axis_index("s") * NC + lax.axis_index("c")
    base = wid * BW
    pltpu.sync_copy(x_hbm.at[pl.ds(wid * IDX_ROWS, IDX_ROWS)], idx_v)

    def fire_gather(g, buf, sem):
        for j in range(NG):
            pltpu.async_copy(
                table_hbm.at[idx_v.at[g * NG + j]],
                buf.at[pl.ds(j * IDXC, IDXC)],
                sem,
            )

    def wait_gather(buf, sem):
        for j in range(NG):
            pltpu.make_async_copy(
                table_hbm.at[idx_v.at[j]],
                buf.at[pl.ds(j * IDXC, IDXC)],
                sem,
            ).wait()

    def scale(buf):
        def row(r, carry):
            for k in range(D_MODEL // LANES):
                sl = pl.ds(k * LANES, LANES)
                buf[r, sl] = buf[r, sl] * SCALE
            return carry

        lax.fori_loop(0, CHUNK, row, 0)

    def chunk(g, carry):
        fire_gather(g, buf0, g0)
        wait_gather(buf0, g0)
        scale(buf0)
        pltpu.sync_copy(buf0, out_hbm.at[pl.ds(base + g * CHUNK, CHUNK)])
        return carry

    lax.fori_loop(0, NCH, chunk, 0)


def kernel(x, lut_weight):
    idx = x.reshape(B_TOTAL // IDXC, IDXC).astype(jnp.int32)
    out = _build()(idx, lut_weight)
    return out.reshape(x.shape[0], x.shape[1], D_MODEL)


# speedup vs baseline: 1.1153x; 1.1153x over previous
import functools
import math

import jax
import jax.numpy as jnp
from jax import lax
from jax.experimental import pallas as pl
from jax.experimental.pallas import tpu as pltpu
from jax.experimental.pallas import tpu_sc as plsc

D_MODEL = 64
SCALE = math.sqrt(D_MODEL)
LANES = 16
NC, NS = 2, 16
NW = NC * NS
B_TOTAL = 4096 * 200
BW = B_TOTAL // NW
IDXC = 128
CHUNK = 256
NG = CHUNK // IDXC
NCH = BW // CHUNK
IDX_ROWS = BW // IDXC
RING = 3


def _emb_body(x_hbm, table_hbm, out_hbm, idx_v, b0, b1, b2, g0, g1, g2,
              s0, s1, s2):
    bufs = (b0, b1, b2)
    gsems = (g0, g1, g2)
    ssems = (s0, s1, s2)
    wid = lax.axis_index("s") * NC + lax.axis_index("c")
    base = wid * BW
    pltpu.sync_copy(x_hbm.at[pl.ds(wid * IDX_ROWS, IDX_ROWS)], idx_v)

    def fire_gather(g, slot):
        for j in range(NG):
            pltpu.async_copy(
                table_hbm.at[idx_v.at[g * NG + j]],
                bufs[slot].at[pl.ds(j * IDXC, IDXC)],
                gsems[slot],
            )

    def wait_gather(slot):
        for j in range(NG):
            pltpu.make_async_copy(
                table_hbm.at[idx_v.at[j]],
                bufs[slot].at[pl.ds(j * IDXC, IDXC)],
                gsems[slot],
            ).wait()

    def fire_store(g, slot):
        pltpu.async_copy(
            bufs[slot], out_hbm.at[pl.ds(base + g * CHUNK, CHUNK)], ssems[slot]
        )

    def wait_store(slot):
        pltpu.make_async_copy(
            bufs[slot], out_hbm.at[pl.ds(base, CHUNK)], ssems[slot]
        ).wait()

    def scale(slot):
        buf = bufs[slot]

        @plsc.parallel_loop(0, CHUNK, unroll=8)
        def _(r):
            for k in range(D_MODEL // LANES):
                sl = pl.ds(k * LANES, LANES)
                buf[r, sl] = buf[r, sl] * SCALE

    fire_gather(0, 0)
    fire_gather(1, 1)

    def outer(t, carry):
        for p in range(RING):
            g = t * RING + p
            nslot = (p + 2) % RING

            @pl.when(g < NCH)
            def _():
                wait_gather(p)
                scale(p)

                @pl.when(g >= 1)
                def _():
                    wait_store(nslot)

                @pl.when(g + 2 < NCH)
                def _():
                    fire_gather(g + 2, nslot)

                fire_store(g, p)

        return carry

    lax.fori_loop(0, (NCH + RING - 1) // RING, outer, 0)
    wait_store((NCH - 1) % RING)


@functools.cache
def _build():
    mesh = plsc.VectorSubcoreMesh(
        core_axis_name="c", subcore_axis_name="s", num_cores=NC, num_subcores=NS
    )
    return functools.partial(
        pl.kernel,
        out_type=jax.ShapeDtypeStruct((B_TOTAL, D_MODEL), jnp.float32),
        mesh=mesh,
        scratch_types=[
            pltpu.VMEM((IDX_ROWS, IDXC), jnp.int32),
            pltpu.VMEM((CHUNK, D_MODEL), jnp.float32),
            pltpu.VMEM((CHUNK, D_MODEL), jnp.float32),
            pltpu.VMEM((CHUNK, D_MODEL), jnp.float32),
            pltpu.SemaphoreType.DMA,
            pltpu.SemaphoreType.DMA,
            pltpu.SemaphoreType.DMA,
            pltpu.SemaphoreType.DMA,
            pltpu.SemaphoreType.DMA,
            pltpu.SemaphoreType.DMA,
        ],
        compiler_params=pltpu.CompilerParams(use_tc_tiling_on_sc=False),
    )(_emb_body)


def kernel(x, lut_weight):
    idx = x.reshape(B_TOTAL // IDXC, IDXC).astype(jnp.int32)
    out = _build()(idx, lut_weight)
    return out.reshape(x.shape[0], x.shape[1], D_MODEL)
